# baseline (speedup 1.0000x reference)
import functools

import jax
import jax.numpy as jnp
from jax import lax
from jax.experimental import pallas as pl
from jax.experimental.pallas import tpu as pltpu
from jax.experimental.pallas import tpu_sc as plsc

S = 2048
DIM = 2048
Q_LORA_RANK = 1536
N_HEADS = 16
HEAD_DIM = 128
ROPE_DIM = 64
INDEX_TOPK = 2048
BLOCK = 128

_NC = 2
_NS = 16
_NW = _NC * _NS
_NV = S // 16
_ROWS_PER_W = S // _NW


def _fwht(x):
    d = x.shape[-1]
    h = 1
    while h < d:
        x = x.reshape(x.shape[:-1] + (d // (2 * h), 2, h))
        a = x[..., 0, :]
        b = x[..., 1, :]
        x = jnp.stack([a + b, a - b], axis=-2)
        x = x.reshape(x.shape[:-3] + (d,))
        h *= 2
    return x


def _rope_interleaved(x, cos, sin, rot_end):
    rot = x[..., :rot_end]
    rest = x[..., rot_end:]
    x1 = rot[..., 0::2]
    x2 = rot[..., 1::2]
    o1 = x1 * cos - x2 * sin
    o2 = x1 * sin + x2 * cos
    out = jnp.stack([o1, o2], axis=-1).reshape(rot.shape)
    return jnp.concatenate([out, rest], axis=-1)


def _block_quant_dequant(x, block=BLOCK):
    shp = x.shape
    xb = x.reshape(shp[:-1] + (shp[-1] // block, block))
    amax = jnp.max(jnp.abs(xb), axis=-1, keepdims=True)
    scale = jnp.maximum(amax, 1e-4) / 448.0
    q = jnp.clip(xb / scale, -448.0, 448.0)
    return (q * scale).reshape(shp)


def _scores(x, qr, wq_b, wk, ln_w, ln_b, w_weights, position_ids):
    softmax_scale = HEAD_DIM ** -0.5
    q = (qr @ wq_b).reshape(S, N_HEADS, HEAD_DIM)
    k = x @ wk
    mu = jnp.mean(k, axis=-1, keepdims=True)
    var = jnp.var(k, axis=-1, keepdims=True)
    k = (k - mu) / jnp.sqrt(var + 1e-6) * ln_w + ln_b
    inv_freq = 1.0 / (10000.0 ** (jnp.arange(0, ROPE_DIM, 2, dtype=jnp.float32) / ROPE_DIM))
    ang = position_ids.astype(jnp.float32)[:, None] * inv_freq[None, :]
    cos = jnp.cos(ang)
    sin = jnp.sin(ang)
    q = _rope_interleaved(q, cos[:, None, :], sin[:, None, :], ROPE_DIM)
    k = _rope_interleaved(k, cos, sin, ROPE_DIM)
    q = _fwht(q) * (HEAD_DIM ** -0.5)
    k = _fwht(k) * (HEAD_DIM ** -0.5)
    q = _block_quant_dequant(q)
    k = _block_quant_dequant(k)
    head_w = x @ w_weights
    scores = jnp.einsum('shd,td->sht', q, k)
    scores = jax.nn.relu(scores)
    scores = jnp.einsum('sht,sh->st', scores, head_w) * softmax_scale
    causal = position_ids[:, None] >= position_ids[None, :]
    return jnp.where(causal, scores, -1e30)


def _to_keys(scores):
    s = scores + 0.0
    u = lax.bitcast_convert_type(s, jnp.int32)
    return jnp.where(s < 0.0, u, ~u & jnp.int32(0x7FFFFFFF))


def _sc_argsort(keys):
    mesh = plsc.VectorSubcoreMesh(core_axis_name="c", subcore_axis_name="s")

    @functools.partial(
        pl.kernel,
        out_type=jax.ShapeDtypeStruct((S, S), jnp.int32),
        mesh=mesh,
        compiler_params=pltpu.CompilerParams(needs_layout_passes=False),
        scratch_types=[
            pltpu.VMEM((S,), jnp.int32),
            pltpu.VMEM((S,), jnp.int32),
            pltpu.VMEM((S,), jnp.int32),
            pltpu.VMEM((S,), jnp.int32),
            pltpu.VMEM((S,), jnp.int32),
            pltpu.VMEM((64 * 16,), jnp.int32),
            pltpu.VMEM((S,), jnp.int32),
        ],
    )
    def sort_kernel(keys_hbm, out_hbm, rowbuf, ka, va, kb, vb, hist, outbuf):
        wid = lax.axis_index("s") * _NC + lax.axis_index("c")
        lanes = lax.iota(jnp.int32, 16)
        ones = jnp.ones((16,), jnp.int32)
        zeros = jnp.zeros((16,), jnp.int32)

        def do_pass(p, shift, src_k, src_v, dst_k, dst_v):
            def zstep(i, c):
                hist[pl.ds(i * 16, 16)] = zeros
                return c

            lax.fori_loop(0, 64, zstep, 0)

            def hstep(v, c):
                if p == 0:
                    key = plsc.load_gather(rowbuf, [v + 128 * lanes])
                else:
                    key = src_k[pl.ds(v * 16, 16)]
                d = lax.shift_right_logical(key, shift) & 63
                plsc.addupdate_scatter(hist, [d * 16 + lanes], ones)
                return c

            lax.fori_loop(0, _NV, hstep, 0)

            def pstep(i, carry):
                h = hist[pl.ds(i * 16, 16)]
                inc = plsc.cumsum(h)
                hist[pl.ds(i * 16, 16)] = inc - h + carry
                return carry + jnp.sum(h)

            lax.fori_loop(0, 64, pstep, jnp.int32(0))

            def mstep(v, c):
                if p == 0:
                    val = v + 128 * lanes
                    key = plsc.load_gather(rowbuf, [val])
                else:
                    key = src_k[pl.ds(v * 16, 16)]
                    val = src_v[pl.ds(v * 16, 16)]
                d = lax.shift_right_logical(key, shift) & 63
                ih = d * 16 + lanes
                r = plsc.load_gather(hist, [ih])
                plsc.addupdate_scatter(hist, [ih], ones)
                if p == 5:
                    plsc.store_scatter(outbuf, [r], val)
                else:
                    addr = (r & 127) * 16 + lax.shift_right_logical(r, 7)
                    plsc.store_scatter(dst_k, [addr], key)
                    plsc.store_scatter(dst_v, [addr], val)
                return c

            lax.fori_loop(0, _NV, mstep, 0)

        def row_body(ri, c):
            row = wid + _NW * ri
            pltpu.sync_copy(keys_hbm.at[row], rowbuf)
            do_pass(0, 0, None, None, kb, vb)
            do_pass(1, 6, kb, vb, ka, va)
            do_pass(2, 12, ka, va, kb, vb)
            do_pass(3, 18, kb, vb, ka, va)
            do_pass(4, 24, ka, va, kb, vb)
            do_pass(5, 30, kb, vb, None, None)
            pltpu.sync_copy(outbuf, out_hbm.at[row])
            return c

        lax.fori_loop(0, _ROWS_PER_W, row_body, 0)

    return sort_kernel(keys)


def kernel(x, qr, wq_b, wk, ln_w, ln_b, w_weights, position_ids):
    scores = _scores(x, qr, wq_b, wk, ln_w, ln_b, w_weights, position_ids)
    keys = _to_keys(scores)
    return _sc_argsort(keys)


# speedup vs baseline: 1.6189x; 1.6189x over previous
import functools

import jax
import jax.numpy as jnp
from jax import lax
from jax.experimental import pallas as pl
from jax.experimental.pallas import tpu as pltpu
from jax.experimental.pallas import tpu_sc as plsc

S = 2048
DIM = 2048
Q_LORA_RANK = 1536
N_HEADS = 16
HEAD_DIM = 128
ROPE_DIM = 64
INDEX_TOPK = 2048
BLOCK = 128

_NC = 2
_NS = 16
_NW = _NC * _NS
_NV = S // 16
_ROWS_PER_W = S // _NW


def _fwht(x):
    d = x.shape[-1]
    h = 1
    while h < d:
        x = x.reshape(x.shape[:-1] + (d // (2 * h), 2, h))
        a = x[..., 0, :]
        b = x[..., 1, :]
        x = jnp.stack([a + b, a - b], axis=-2)
        x = x.reshape(x.shape[:-3] + (d,))
        h *= 2
    return x


def _rope_interleaved(x, cos, sin, rot_end):
    rot = x[..., :rot_end]
    rest = x[..., rot_end:]
    x1 = rot[..., 0::2]
    x2 = rot[..., 1::2]
    o1 = x1 * cos - x2 * sin
    o2 = x1 * sin + x2 * cos
    out = jnp.stack([o1, o2], axis=-1).reshape(rot.shape)
    return jnp.concatenate([out, rest], axis=-1)


def _block_quant_dequant(x, block=BLOCK):
    shp = x.shape
    xb = x.reshape(shp[:-1] + (shp[-1] // block, block))
    amax = jnp.max(jnp.abs(xb), axis=-1, keepdims=True)
    scale = jnp.maximum(amax, 1e-4) / 448.0
    q = jnp.clip(xb / scale, -448.0, 448.0)
    return (q * scale).reshape(shp)


def _scores(x, qr, wq_b, wk, ln_w, ln_b, w_weights, position_ids):
    softmax_scale = HEAD_DIM ** -0.5
    q = (qr @ wq_b).reshape(S, N_HEADS, HEAD_DIM)
    k = x @ wk
    mu = jnp.mean(k, axis=-1, keepdims=True)
    var = jnp.var(k, axis=-1, keepdims=True)
    k = (k - mu) / jnp.sqrt(var + 1e-6) * ln_w + ln_b
    inv_freq = 1.0 / (10000.0 ** (jnp.arange(0, ROPE_DIM, 2, dtype=jnp.float32) / ROPE_DIM))
    ang = position_ids.astype(jnp.float32)[:, None] * inv_freq[None, :]
    cos = jnp.cos(ang)
    sin = jnp.sin(ang)
    q = _rope_interleaved(q, cos[:, None, :], sin[:, None, :], ROPE_DIM)
    k = _rope_interleaved(k, cos, sin, ROPE_DIM)
    q = _fwht(q) * (HEAD_DIM ** -0.5)
    k = _fwht(k) * (HEAD_DIM ** -0.5)
    q = _block_quant_dequant(q)
    k = _block_quant_dequant(k)
    head_w = x @ w_weights
    scores = jnp.einsum('shd,td->sht', q, k)
    scores = jax.nn.relu(scores)
    scores = jnp.einsum('sht,sh->st', scores, head_w) * softmax_scale
    causal = position_ids[:, None] >= position_ids[None, :]
    return jnp.where(causal, scores, -1e30)


def _to_keys(scores):
    s = scores + 0.0
    u = lax.bitcast_convert_type(s, jnp.int32)
    return jnp.where(s < 0.0, u, ~u & jnp.int32(0x7FFFFFFF))


def _sc_argsort(keys):
    mesh = plsc.VectorSubcoreMesh(core_axis_name="c", subcore_axis_name="s")

    @functools.partial(
        pl.kernel,
        out_type=jax.ShapeDtypeStruct((S, S), jnp.int32),
        mesh=mesh,
        compiler_params=pltpu.CompilerParams(needs_layout_passes=False),
        scratch_types=[
            pltpu.VMEM((2080,), jnp.int32),
            pltpu.VMEM((2080,), jnp.int32),
            pltpu.VMEM((2080,), jnp.int32),
            pltpu.VMEM((2080,), jnp.int32),
            pltpu.VMEM((2080,), jnp.int32),
            pltpu.VMEM((64 * 16,), jnp.int32),
            pltpu.VMEM((2080,), jnp.int32),
        ],
    )
    def sort_kernel(keys_hbm, out_hbm, rowbuf, ka, va, kb, vb, hist, outbuf):
        wid = lax.axis_index("s") * _NC + lax.axis_index("c")
        lanes = lax.iota(jnp.int32, 16)
        ones = jnp.ones((16,), jnp.int32)
        zeros = jnp.zeros((16,), jnp.int32)

        def do_pass(p, shift, bins, nvo, src_k, src_v, dst_k, dst_v):
            def zstep(i, c):
                hist[pl.ds(i * 16, 16)] = zeros
                return c

            lax.fori_loop(0, bins, zstep, 0)

            def hstep(v, c):
                key = plsc.load_gather(src_k, [v + nvo * lanes])
                d = lax.shift_right_logical(key, shift) & 63
                plsc.addupdate_scatter(hist, [d * 16 + lanes], ones)
                return c

            lax.fori_loop(0, nvo, hstep, 0)

            def pstep(i, carry):
                h = hist[pl.ds(i * 16, 16)]
                inc = plsc.cumsum(h)
                hist[pl.ds(i * 16, 16)] = inc - h + carry
                return carry + jnp.sum(h)

            lax.fori_loop(0, bins, pstep, jnp.int32(0))

            def mstep(v, c):
                ii = v + nvo * lanes
                key = plsc.load_gather(src_k, [ii])
                if p == 0:
                    val = ii
                else:
                    val = plsc.load_gather(src_v, [ii])
                d = lax.shift_right_logical(key, shift) & 63
                ih = d * 16 + lanes
                r = plsc.load_gather(hist, [ih])
                plsc.addupdate_scatter(hist, [ih], ones)
                if p == 5:
                    plsc.store_scatter(outbuf, [r], val)
                else:
                    plsc.store_scatter(dst_k, [r], key)
                    plsc.store_scatter(dst_v, [r], val)
                return c

            lax.fori_loop(0, nvo, mstep, 0)

        def row_body(ri, c):
            row = wid + _NW * ri
            n = row + 1
            nvr = lax.shift_right_logical(n + 15, 4)
            nvo = nvr | 1
            pltpu.sync_copy(keys_hbm.at[row], rowbuf.at[pl.ds(0, S)])

            def fill(v, c):
                outbuf[pl.ds(v * 16, 16)] = v * 16 + lanes
                return c

            lax.fori_loop(nvo, _NV, fill, 0)

            def padstep(v, c):
                a = v * 16 + lanes
                orig = rowbuf[pl.ds(v * 16, 16)]
                rowbuf[pl.ds(v * 16, 16)] = jnp.where(
                    a < n, orig, jnp.int32(-65536) + a)
                return c

            lax.fori_loop(lax.shift_right_logical(n, 4), nvo, padstep, 0)

            do_pass(0, 0, 64, nvo, rowbuf, None, ka, va)
            do_pass(1, 6, 64, nvo, ka, va, kb, vb)
            do_pass(2, 12, 64, nvo, kb, vb, ka, va)
            do_pass(3, 18, 64, nvo, ka, va, kb, vb)
            do_pass(4, 24, 64, nvo, kb, vb, ka, va)
            do_pass(5, 30, 4, nvo, ka, va, None, None)
            pltpu.sync_copy(outbuf.at[pl.ds(0, S)], out_hbm.at[row])
            return c

        lax.fori_loop(0, _ROWS_PER_W, row_body, 0)

    return sort_kernel(keys)


def kernel(x, qr, wq_b, wk, ln_w, ln_b, w_weights, position_ids):
    scores = _scores(x, qr, wq_b, wk, ln_w, ln_b, w_weights, position_ids)
    keys = _to_keys(scores)
    return _sc_argsort(keys)


# speedup vs baseline: 1.7371x; 1.0730x over previous
import functools

import jax
import jax.numpy as jnp
from jax import lax
from jax.experimental import pallas as pl
from jax.experimental.pallas import tpu as pltpu
from jax.experimental.pallas import tpu_sc as plsc

S = 2048
DIM = 2048
Q_LORA_RANK = 1536
N_HEADS = 16
HEAD_DIM = 128
ROPE_DIM = 64
INDEX_TOPK = 2048
BLOCK = 128

_NC = 2
_NS = 16
_NW = _NC * _NS
_NV = S // 16
_ROWS_PER_W = S // _NW


def _fwht(x):
    d = x.shape[-1]
    h = 1
    while h < d:
        x = x.reshape(x.shape[:-1] + (d // (2 * h), 2, h))
        a = x[..., 0, :]
        b = x[..., 1, :]
        x = jnp.stack([a + b, a - b], axis=-2)
        x = x.reshape(x.shape[:-3] + (d,))
        h *= 2
    return x


def _rope_interleaved(x, cos, sin, rot_end):
    rot = x[..., :rot_end]
    rest = x[..., rot_end:]
    x1 = rot[..., 0::2]
    x2 = rot[..., 1::2]
    o1 = x1 * cos - x2 * sin
    o2 = x1 * sin + x2 * cos
    out = jnp.stack([o1, o2], axis=-1).reshape(rot.shape)
    return jnp.concatenate([out, rest], axis=-1)


def _block_quant_dequant(x, block=BLOCK):
    shp = x.shape
    xb = x.reshape(shp[:-1] + (shp[-1] // block, block))
    amax = jnp.max(jnp.abs(xb), axis=-1, keepdims=True)
    scale = jnp.maximum(amax, 1e-4) / 448.0
    q = jnp.clip(xb / scale, -448.0, 448.0)
    return (q * scale).reshape(shp)


def _scores(x, qr, wq_b, wk, ln_w, ln_b, w_weights, position_ids):
    softmax_scale = HEAD_DIM ** -0.5
    q = (qr @ wq_b).reshape(S, N_HEADS, HEAD_DIM)
    k = x @ wk
    mu = jnp.mean(k, axis=-1, keepdims=True)
    var = jnp.var(k, axis=-1, keepdims=True)
    k = (k - mu) / jnp.sqrt(var + 1e-6) * ln_w + ln_b
    inv_freq = 1.0 / (10000.0 ** (jnp.arange(0, ROPE_DIM, 2, dtype=jnp.float32) / ROPE_DIM))
    ang = position_ids.astype(jnp.float32)[:, None] * inv_freq[None, :]
    cos = jnp.cos(ang)
    sin = jnp.sin(ang)
    q = _rope_interleaved(q, cos[:, None, :], sin[:, None, :], ROPE_DIM)
    k = _rope_interleaved(k, cos, sin, ROPE_DIM)
    q = _fwht(q) * (HEAD_DIM ** -0.5)
    k = _fwht(k) * (HEAD_DIM ** -0.5)
    q = _block_quant_dequant(q)
    k = _block_quant_dequant(k)
    head_w = x @ w_weights
    scores = jnp.einsum('shd,td->sht', q, k)
    scores = jax.nn.relu(scores)
    scores = jnp.einsum('sht,sh->st', scores, head_w) * softmax_scale
    causal = position_ids[:, None] >= position_ids[None, :]
    return jnp.where(causal, scores, -1e30)


def _to_keys(scores):
    s = scores + 0.0
    u = lax.bitcast_convert_type(s, jnp.int32)
    return jnp.where(s < 0.0, u, ~u & jnp.int32(0x7FFFFFFF))


def _sc_argsort(keys):
    mesh = plsc.VectorSubcoreMesh(core_axis_name="c", subcore_axis_name="s")

    @functools.partial(
        pl.kernel,
        out_type=jax.ShapeDtypeStruct((S, S), jnp.int32),
        mesh=mesh,
        compiler_params=pltpu.CompilerParams(needs_layout_passes=False),
        scratch_types=[
            pltpu.VMEM((2080,), jnp.int32),
            pltpu.VMEM((2080,), jnp.int32),
            pltpu.VMEM((2080,), jnp.int32),
            pltpu.VMEM((2080,), jnp.int32),
            pltpu.VMEM((2080,), jnp.int32),
            pltpu.VMEM((2080,), jnp.int32),
            pltpu.VMEM((2080,), jnp.int32),
            pltpu.VMEM((2080,), jnp.int32),
            pltpu.VMEM((2080,), jnp.int32),
            pltpu.VMEM((2080,), jnp.int32),
            pltpu.VMEM((64 * 16,), jnp.int32),
            pltpu.VMEM((64 * 16,), jnp.int32),
            pltpu.VMEM((2080,), jnp.int32),
            pltpu.VMEM((2080,), jnp.int32),
        ],
    )
    def sort_kernel(keys_hbm, out_hbm, rowbuf0, rowbuf1, ka0, va0, kb0, vb0,
                    ka1, va1, kb1, vb1, hist0, hist1, outbuf0, outbuf1):
        wid = lax.axis_index("s") * _NC + lax.axis_index("c")
        lanes = lax.iota(jnp.int32, 16)
        ones = jnp.ones((16,), jnp.int32)
        zeros = jnp.zeros((16,), jnp.int32)
        hists = (hist0, hist1)
        outbufs = (outbuf0, outbuf1)

        def do_pass(p, shift, bins, nvo, srcs, dsts):
            def zstep(i, c):
                for j in range(2):
                    hists[j][pl.ds(i * 16, 16)] = zeros
                return c

            lax.fori_loop(0, bins, zstep, 0)

            def hstep(v, c):
                for j in range(2):
                    key = plsc.load_gather(srcs[j][0], [v + nvo * lanes])
                    d = lax.shift_right_logical(key, shift) & 63
                    plsc.addupdate_scatter(hists[j], [d * 16 + lanes], ones)
                return c

            lax.fori_loop(0, nvo, hstep, 0)

            def pstep(i, carries):
                outs = []
                for j in range(2):
                    h = hists[j][pl.ds(i * 16, 16)]
                    inc = plsc.cumsum(h)
                    hists[j][pl.ds(i * 16, 16)] = inc - h + carries[j]
                    outs.append(carries[j] + inc[15])
                return tuple(outs)

            lax.fori_loop(0, bins, pstep, (jnp.int32(0), jnp.int32(0)))

            def mstep(v, c):
                for j in range(2):
                    ii = v + nvo * lanes
                    key = plsc.load_gather(srcs[j][0], [ii])
                    if p == 0:
                        val = ii
                    else:
                        val = plsc.load_gather(srcs[j][1], [ii])
                    d = lax.shift_right_logical(key, shift) & 63
                    ih = d * 16 + lanes
                    r = plsc.load_gather(hists[j], [ih])
                    plsc.addupdate_scatter(hists[j], [ih], ones)
                    if p == 5:
                        plsc.store_scatter(outbufs[j], [r], val)
                    else:
                        plsc.store_scatter(dsts[j][0], [r], key)
                        plsc.store_scatter(dsts[j][1], [r], val)
                return c

            lax.fori_loop(0, nvo, mstep, 0)

        def row_body(ri, c):
            row0 = wid + _NW * 2 * ri
            row1 = row0 + _NW
            rows = (row0, row1)
            rowbufs = ((rowbuf0, None), (rowbuf1, None))
            n1 = row1 + 1
            nvo = lax.shift_right_logical(n1 + 15, 4) | 1
            pltpu.sync_copy(keys_hbm.at[row0], rowbuf0.at[pl.ds(0, S)])
            pltpu.sync_copy(keys_hbm.at[row1], rowbuf1.at[pl.ds(0, S)])

            def fill(v, c):
                for j in range(2):
                    outbufs[j][pl.ds(v * 16, 16)] = v * 16 + lanes
                return c

            lax.fori_loop(nvo, _NV, fill, 0)

            def padstep(v, c):
                for j in range(2):
                    a = v * 16 + lanes
                    rb = rowbufs[j][0]
                    orig = rb[pl.ds(v * 16, 16)]
                    rb[pl.ds(v * 16, 16)] = jnp.where(
                        a < rows[j] + 1, orig, jnp.int32(-65536) + a)
                return c

            lax.fori_loop(lax.shift_right_logical(row0 + 1, 4), nvo,
                          padstep, 0)

            a_bufs = ((ka0, va0), (ka1, va1))
            b_bufs = ((kb0, vb0), (kb1, vb1))
            do_pass(0, 0, 64, nvo, rowbufs, a_bufs)
            do_pass(1, 6, 64, nvo, a_bufs, b_bufs)
            do_pass(2, 12, 64, nvo, b_bufs, a_bufs)
            do_pass(3, 18, 64, nvo, a_bufs, b_bufs)
            do_pass(4, 24, 64, nvo, b_bufs, a_bufs)
            do_pass(5, 30, 4, nvo, a_bufs, None)
            pltpu.sync_copy(outbuf0.at[pl.ds(0, S)], out_hbm.at[row0])
            pltpu.sync_copy(outbuf1.at[pl.ds(0, S)], out_hbm.at[row1])
            return c

        lax.fori_loop(0, _ROWS_PER_W // 2, row_body, 0)

    return sort_kernel(keys)


def kernel(x, qr, wq_b, wk, ln_w, ln_b, w_weights, position_ids):
    scores = _scores(x, qr, wq_b, wk, ln_w, ln_b, w_weights, position_ids)
    keys = _to_keys(scores)
    return _sc_argsort(keys)


# speedup vs baseline: 1.9180x; 1.1042x over previous
import functools

import jax
import jax.numpy as jnp
from jax import lax
from jax.experimental import pallas as pl
from jax.experimental.pallas import tpu as pltpu
from jax.experimental.pallas import tpu_sc as plsc

S = 2048
DIM = 2048
Q_LORA_RANK = 1536
N_HEADS = 16
HEAD_DIM = 128
ROPE_DIM = 64
INDEX_TOPK = 2048
BLOCK = 128

_NC = 2
_NS = 16
_NW = _NC * _NS
_NV = S // 16
_ROWS_PER_W = S // _NW


def _fwht(x):
    d = x.shape[-1]
    h = 1
    while h < d:
        x = x.reshape(x.shape[:-1] + (d // (2 * h), 2, h))
        a = x[..., 0, :]
        b = x[..., 1, :]
        x = jnp.stack([a + b, a - b], axis=-2)
        x = x.reshape(x.shape[:-3] + (d,))
        h *= 2
    return x


def _rope_interleaved(x, cos, sin, rot_end):
    rot = x[..., :rot_end]
    rest = x[..., rot_end:]
    x1 = rot[..., 0::2]
    x2 = rot[..., 1::2]
    o1 = x1 * cos - x2 * sin
    o2 = x1 * sin + x2 * cos
    out = jnp.stack([o1, o2], axis=-1).reshape(rot.shape)
    return jnp.concatenate([out, rest], axis=-1)


def _block_quant_dequant(x, block=BLOCK):
    shp = x.shape
    xb = x.reshape(shp[:-1] + (shp[-1] // block, block))
    amax = jnp.max(jnp.abs(xb), axis=-1, keepdims=True)
    scale = jnp.maximum(amax, 1e-4) / 448.0
    q = jnp.clip(xb / scale, -448.0, 448.0)
    return (q * scale).reshape(shp)


def _scores(x, qr, wq_b, wk, ln_w, ln_b, w_weights, position_ids):
    softmax_scale = HEAD_DIM ** -0.5
    q = (qr @ wq_b).reshape(S, N_HEADS, HEAD_DIM)
    k = x @ wk
    mu = jnp.mean(k, axis=-1, keepdims=True)
    var = jnp.var(k, axis=-1, keepdims=True)
    k = (k - mu) / jnp.sqrt(var + 1e-6) * ln_w + ln_b
    inv_freq = 1.0 / (10000.0 ** (jnp.arange(0, ROPE_DIM, 2, dtype=jnp.float32) / ROPE_DIM))
    ang = position_ids.astype(jnp.float32)[:, None] * inv_freq[None, :]
    cos = jnp.cos(ang)
    sin = jnp.sin(ang)
    q = _rope_interleaved(q, cos[:, None, :], sin[:, None, :], ROPE_DIM)
    k = _rope_interleaved(k, cos, sin, ROPE_DIM)
    q = _fwht(q) * (HEAD_DIM ** -0.5)
    k = _fwht(k) * (HEAD_DIM ** -0.5)
    q = _block_quant_dequant(q)
    k = _block_quant_dequant(k)
    head_w = x @ w_weights
    scores = jnp.einsum('shd,td->sht', q, k)
    scores = jax.nn.relu(scores)
    scores = jnp.einsum('sht,sh->st', scores, head_w) * softmax_scale
    causal = position_ids[:, None] >= position_ids[None, :]
    return jnp.where(causal, scores, -1e30)


def _to_keys(scores):
    s = scores + 0.0
    u = lax.bitcast_convert_type(s, jnp.int32)
    return jnp.where(s < 0.0, u, ~u & jnp.int32(0x7FFFFFFF))


def _lane_iota(shape):
    return lax.broadcasted_iota(jnp.int32, shape, len(shape) - 1)


def _rope_roll(x, c_tab, sg_tab):
    li = _lane_iota(x.shape)
    even = (li & 1) == 0
    ax = len(x.shape) - 1
    xs = jnp.where(even, pltpu.roll(x, HEAD_DIM - 1, ax),
                   pltpu.roll(x, 1, ax))
    ro = x * c_tab + xs * sg_tab
    return jnp.where(li < ROPE_DIM, ro, x)


def _fwht_roll(x):
    li = _lane_iota(x.shape)
    ax = len(x.shape) - 1
    for h in (1, 2, 4, 8, 16, 32, 64):
        even = (li & h) == 0
        partner = jnp.where(even, pltpu.roll(x, HEAD_DIM - h, ax),
                            pltpu.roll(x, h, ax))
        x = jnp.where(even, x + partner, partner - x)
    return x


def _quant(x):
    amax = jnp.max(jnp.abs(x), axis=-1, keepdims=True)
    scale = jnp.maximum(amax, 1e-4) / 448.0
    return jnp.clip(x / scale, -448.0, 448.0) * scale


def _rope_tables(position_ids):
    inv_freq = 1.0 / (10000.0 ** (jnp.arange(0, ROPE_DIM, 2, dtype=jnp.float32) / ROPE_DIM))
    ang = position_ids.astype(jnp.float32)[:, None] * inv_freq[None, :]
    cos = jnp.cos(ang)
    sin = jnp.sin(ang)
    c_tab = jnp.repeat(cos, 2, axis=1)
    sg_tab = jnp.stack([-sin, sin], axis=-1).reshape(S, ROPE_DIM)
    pad = jnp.zeros((S, HEAD_DIM - ROPE_DIM), jnp.float32)
    c_tab = jnp.concatenate([c_tab, 1.0 + pad], axis=1)
    sg_tab = jnp.concatenate([sg_tab, pad], axis=1)
    return c_tab, sg_tab


def _kq_body(x_ref, wk_ref, ww_ref, c_ref, sg_ref, ln_w_ref, ln_b_ref,
             k_out, hw_out):
    x = x_ref[...]
    k = jnp.dot(x, wk_ref[...], preferred_element_type=jnp.float32)
    mu = jnp.mean(k, axis=-1, keepdims=True)
    var = jnp.var(k, axis=-1, keepdims=True)
    k = (k - mu) / jnp.sqrt(var + 1e-6) * ln_w_ref[...] + ln_b_ref[...]
    k = _rope_roll(k, c_ref[...], sg_ref[...])
    k = _fwht_roll(k) * (HEAD_DIM ** -0.5)
    k_out[...] = _quant(k)
    hw_out[...] = jnp.dot(x, ww_ref[...], preferred_element_type=jnp.float32)


def _q_body(qr_ref, wq_ref, c_ref, sg_ref, q_out):
    q = jnp.dot(qr_ref[...], wq_ref[...], preferred_element_type=jnp.float32)
    q = q.reshape(q.shape[0], N_HEADS, HEAD_DIM)
    q = _rope_roll(q, c_ref[...][:, None, :], sg_ref[...][:, None, :])
    q = _fwht_roll(q) * (HEAD_DIM ** -0.5)
    q = _quant(q)
    q_out[...] = q.reshape(q.shape[0], N_HEADS * HEAD_DIM)


_SBLK = 256


def _score_body(q_ref, k_ref, hw_ref, keys_out):
    i = pl.program_id(0)
    q = q_ref[...].reshape(_SBLK, N_HEADS, HEAD_DIM)
    k = k_ref[...]
    hw = hw_ref[...]
    acc = jnp.zeros((_SBLK, S), jnp.float32)
    for h in range(N_HEADS):
        sc = lax.dot_general(q[:, h, :], k, (((1,), (1,)), ((), ())),
                             preferred_element_type=jnp.float32,
                             precision=lax.Precision.HIGHEST)
        acc = acc + jnp.maximum(sc, 0.0) * hw[:, h:h + 1]
    acc = acc * (HEAD_DIM ** -0.5)
    srow = i * _SBLK + lax.broadcasted_iota(jnp.int32, (_SBLK, S), 0)
    tcol = lax.broadcasted_iota(jnp.int32, (_SBLK, S), 1)
    acc = jnp.where(srow >= tcol, acc, -1e30)
    keys_out[...] = _to_keys(acc)


def _scores_tc(x, qr, wq_b, wk, ln_w, ln_b, w_weights, position_ids):
    c_tab, sg_tab = _rope_tables(position_ids)
    ln_w2 = ln_w.reshape(1, HEAD_DIM)
    ln_b2 = ln_b.reshape(1, HEAD_DIM)
    kb = 512
    k_hat, hw = pl.pallas_call(
        _kq_body,
        grid=(S // kb,),
        in_specs=[
            pl.BlockSpec((kb, DIM), lambda i: (i, 0)),
            pl.BlockSpec((DIM, HEAD_DIM), lambda i: (0, 0)),
            pl.BlockSpec((DIM, N_HEADS), lambda i: (0, 0)),
            pl.BlockSpec((kb, HEAD_DIM), lambda i: (i, 0)),
            pl.BlockSpec((kb, HEAD_DIM), lambda i: (i, 0)),
            pl.BlockSpec((1, HEAD_DIM), lambda i: (0, 0)),
            pl.BlockSpec((1, HEAD_DIM), lambda i: (0, 0)),
        ],
        out_specs=[
            pl.BlockSpec((kb, HEAD_DIM), lambda i: (i, 0)),
            pl.BlockSpec((kb, N_HEADS), lambda i: (i, 0)),
        ],
        out_shape=[
            jax.ShapeDtypeStruct((S, HEAD_DIM), jnp.float32),
            jax.ShapeDtypeStruct((S, N_HEADS), jnp.float32),
        ],
    )(x, wk, w_weights, c_tab, sg_tab, ln_w2, ln_b2)

    qb = 512
    q_hat = pl.pallas_call(
        _q_body,
        grid=(S // qb,),
        in_specs=[
            pl.BlockSpec((qb, Q_LORA_RANK), lambda i: (i, 0)),
            pl.BlockSpec((Q_LORA_RANK, N_HEADS * HEAD_DIM), lambda i: (0, 0)),
            pl.BlockSpec((qb, HEAD_DIM), lambda i: (i, 0)),
            pl.BlockSpec((qb, HEAD_DIM), lambda i: (i, 0)),
        ],
        out_specs=pl.BlockSpec((qb, N_HEADS * HEAD_DIM), lambda i: (i, 0)),
        out_shape=jax.ShapeDtypeStruct((S, N_HEADS * HEAD_DIM), jnp.float32),
    )(qr, wq_b, c_tab, sg_tab)

    keys = pl.pallas_call(
        _score_body,
        grid=(S // _SBLK,),
        in_specs=[
            pl.BlockSpec((_SBLK, N_HEADS * HEAD_DIM), lambda i: (i, 0)),
            pl.BlockSpec((S, HEAD_DIM), lambda i: (0, 0)),
            pl.BlockSpec((_SBLK, N_HEADS), lambda i: (i, 0)),
        ],
        out_specs=pl.BlockSpec((_SBLK, S), lambda i: (i, 0)),
        out_shape=jax.ShapeDtypeStruct((S, S), jnp.int32),
    )(q_hat, k_hat, hw)
    return keys


def _sc_argsort(keys):
    mesh = plsc.VectorSubcoreMesh(core_axis_name="c", subcore_axis_name="s")
    B = 4

    @functools.partial(
        pl.kernel,
        out_type=jax.ShapeDtypeStruct((S, S), jnp.int32),
        mesh=mesh,
        compiler_params=pltpu.CompilerParams(needs_layout_passes=False),
        scratch_types=(
            [pltpu.VMEM((2080,), jnp.int32) for _ in range(B)]
            + [pltpu.VMEM((2080,), jnp.int32) for _ in range(4 * B)]
            + [pltpu.VMEM((64 * 16,), jnp.int32) for _ in range(B)]
            + [pltpu.VMEM((2080,), jnp.int32) for _ in range(B)]
        ),
    )
    def sort_kernel(keys_hbm, out_hbm, *scr):
        rowbufs = scr[0:B]
        ka = scr[B:2 * B]
        va = scr[2 * B:3 * B]
        kb = scr[3 * B:4 * B]
        vb = scr[4 * B:5 * B]
        hists = scr[5 * B:6 * B]
        outbufs = scr[6 * B:7 * B]
        a_bufs = tuple(zip(ka, va))
        b_bufs = tuple(zip(kb, vb))
        src0 = tuple((rb, None) for rb in rowbufs)
        wid = lax.axis_index("s") * _NC + lax.axis_index("c")
        lanes = lax.iota(jnp.int32, 16)
        ones = jnp.ones((16,), jnp.int32)
        zeros = jnp.zeros((16,), jnp.int32)

        def do_pass(p, shift, bins, nvo, srcs, dsts):
            def zstep(i, c):
                for j in range(B):
                    hists[j][pl.ds(i * 16, 16)] = zeros
                return c

            lax.fori_loop(0, bins, zstep, 0)

            def hstep(v, c):
                for j in range(B):
                    key = plsc.load_gather(srcs[j][0], [v + nvo * lanes])
                    d = lax.shift_right_logical(key, shift) & 63
                    plsc.addupdate_scatter(hists[j], [d * 16 + lanes], ones)
                return c

            lax.fori_loop(0, nvo, hstep, 0)

            def pstep(i, carries):
                outs = []
                for j in range(B):
                    h = hists[j][pl.ds(i * 16, 16)]
                    inc = plsc.cumsum(h)
                    hists[j][pl.ds(i * 16, 16)] = inc - h + carries[j]
                    outs.append(carries[j] + inc[15])
                return tuple(outs)

            lax.fori_loop(0, bins, pstep, (jnp.int32(0),) * B)

            def mstep(v, c):
                for j in range(B):
                    ii = v + nvo * lanes
                    key = plsc.load_gather(srcs[j][0], [ii])
                    if p == 0:
                        val = ii
                    else:
                        val = plsc.load_gather(srcs[j][1], [ii])
                    d = lax.shift_right_logical(key, shift) & 63
                    ih = d * 16 + lanes
                    r = plsc.load_gather(hists[j], [ih])
                    plsc.addupdate_scatter(hists[j], [ih], ones)
                    if p == 5:
                        plsc.store_scatter(outbufs[j], [r], val)
                    else:
                        plsc.store_scatter(dsts[j][0], [r], key)
                        plsc.store_scatter(dsts[j][1], [r], val)
                return c

            lax.fori_loop(0, nvo, mstep, 0)

        def row_body(ri, c):
            rows = tuple(wid + _NW * (B * ri + j) for j in range(B))
            nvo = lax.shift_right_logical(rows[B - 1] + 16, 4) | 1
            for j in range(B):
                pltpu.sync_copy(keys_hbm.at[rows[j]],
                                rowbufs[j].at[pl.ds(0, S)])

            def fill(v, c):
                for j in range(B):
                    outbufs[j][pl.ds(v * 16, 16)] = v * 16 + lanes
                return c

            lax.fori_loop(nvo, _NV, fill, 0)

            def padstep(v, c):
                for j in range(B):
                    a = v * 16 + lanes
                    rb = rowbufs[j]
                    orig = rb[pl.ds(v * 16, 16)]
                    rb[pl.ds(v * 16, 16)] = jnp.where(
                        a < rows[j] + 1, orig, jnp.int32(-65536) + a)
                return c

            lax.fori_loop(lax.shift_right_logical(rows[0] + 1, 4), nvo,
                          padstep, 0)

            do_pass(0, 0, 64, nvo, src0, a_bufs)
            do_pass(1, 6, 64, nvo, a_bufs, b_bufs)
            do_pass(2, 12, 64, nvo, b_bufs, a_bufs)
            do_pass(3, 18, 64, nvo, a_bufs, b_bufs)
            do_pass(4, 24, 64, nvo, b_bufs, a_bufs)
            do_pass(5, 30, 4, nvo, a_bufs, None)
            for j in range(B):
                pltpu.sync_copy(outbufs[j].at[pl.ds(0, S)],
                                out_hbm.at[rows[j]])
            return c

        lax.fori_loop(0, _ROWS_PER_W // B, row_body, 0)

    return sort_kernel(keys)


def kernel(x, qr, wq_b, wk, ln_w, ln_b, w_weights, position_ids):
    scores = _scores(x, qr, wq_b, wk, ln_w, ln_b, w_weights, position_ids)
    keys = _to_keys(scores)
    return _sc_argsort(keys)


# speedup vs baseline: 1.9685x; 1.0263x over previous
import functools

import jax
import jax.numpy as jnp
from jax import lax
from jax.experimental import pallas as pl
from jax.experimental.pallas import tpu as pltpu
from jax.experimental.pallas import tpu_sc as plsc

S = 2048
DIM = 2048
Q_LORA_RANK = 1536
N_HEADS = 16
HEAD_DIM = 128
ROPE_DIM = 64
INDEX_TOPK = 2048
BLOCK = 128

_NC = 2
_NS = 16
_NW = _NC * _NS
_NV = S // 16
_ROWS_PER_W = S // _NW


def _fwht(x):
    d = x.shape[-1]
    h = 1
    while h < d:
        x = x.reshape(x.shape[:-1] + (d // (2 * h), 2, h))
        a = x[..., 0, :]
        b = x[..., 1, :]
        x = jnp.stack([a + b, a - b], axis=-2)
        x = x.reshape(x.shape[:-3] + (d,))
        h *= 2
    return x


def _rope_interleaved(x, cos, sin, rot_end):
    rot = x[..., :rot_end]
    rest = x[..., rot_end:]
    x1 = rot[..., 0::2]
    x2 = rot[..., 1::2]
    o1 = x1 * cos - x2 * sin
    o2 = x1 * sin + x2 * cos
    out = jnp.stack([o1, o2], axis=-1).reshape(rot.shape)
    return jnp.concatenate([out, rest], axis=-1)


def _block_quant_dequant(x, block=BLOCK):
    shp = x.shape
    xb = x.reshape(shp[:-1] + (shp[-1] // block, block))
    amax = jnp.max(jnp.abs(xb), axis=-1, keepdims=True)
    scale = jnp.maximum(amax, 1e-4) / 448.0
    q = jnp.clip(xb / scale, -448.0, 448.0)
    return (q * scale).reshape(shp)


def _scores(x, qr, wq_b, wk, ln_w, ln_b, w_weights, position_ids):
    softmax_scale = HEAD_DIM ** -0.5
    q = (qr @ wq_b).reshape(S, N_HEADS, HEAD_DIM)
    k = x @ wk
    mu = jnp.mean(k, axis=-1, keepdims=True)
    var = jnp.var(k, axis=-1, keepdims=True)
    k = (k - mu) / jnp.sqrt(var + 1e-6) * ln_w + ln_b
    inv_freq = 1.0 / (10000.0 ** (jnp.arange(0, ROPE_DIM, 2, dtype=jnp.float32) / ROPE_DIM))
    ang = position_ids.astype(jnp.float32)[:, None] * inv_freq[None, :]
    cos = jnp.cos(ang)
    sin = jnp.sin(ang)
    q = _rope_interleaved(q, cos[:, None, :], sin[:, None, :], ROPE_DIM)
    k = _rope_interleaved(k, cos, sin, ROPE_DIM)
    q = _fwht(q) * (HEAD_DIM ** -0.5)
    k = _fwht(k) * (HEAD_DIM ** -0.5)
    q = _block_quant_dequant(q)
    k = _block_quant_dequant(k)
    head_w = x @ w_weights
    scores = jnp.einsum('shd,td->sht', q, k)
    scores = jax.nn.relu(scores)
    scores = jnp.einsum('sht,sh->st', scores, head_w) * softmax_scale
    causal = position_ids[:, None] >= position_ids[None, :]
    return jnp.where(causal, scores, -1e30)


def _to_keys(scores):
    s = scores + 0.0
    u = lax.bitcast_convert_type(s, jnp.int32)
    return jnp.where(s < 0.0, u, ~u & jnp.int32(0x7FFFFFFF))


def _lane_iota(shape):
    return lax.broadcasted_iota(jnp.int32, shape, len(shape) - 1)


def _rope_roll(x, c_tab, sg_tab):
    li = _lane_iota(x.shape)
    even = (li & 1) == 0
    ax = len(x.shape) - 1
    xs = jnp.where(even, pltpu.roll(x, HEAD_DIM - 1, ax),
                   pltpu.roll(x, 1, ax))
    ro = x * c_tab + xs * sg_tab
    return jnp.where(li < ROPE_DIM, ro, x)


def _fwht_roll(x):
    li = _lane_iota(x.shape)
    ax = len(x.shape) - 1
    for h in (1, 2, 4, 8, 16, 32, 64):
        even = (li & h) == 0
        partner = jnp.where(even, pltpu.roll(x, HEAD_DIM - h, ax),
                            pltpu.roll(x, h, ax))
        x = jnp.where(even, x + partner, partner - x)
    return x


def _quant(x):
    amax = jnp.max(jnp.abs(x), axis=-1, keepdims=True)
    scale = jnp.maximum(amax, 1e-4) / 448.0
    return jnp.clip(x / scale, -448.0, 448.0) * scale


def _rope_tables(position_ids):
    inv_freq = 1.0 / (10000.0 ** (jnp.arange(0, ROPE_DIM, 2, dtype=jnp.float32) / ROPE_DIM))
    ang = position_ids.astype(jnp.float32)[:, None] * inv_freq[None, :]
    cos = jnp.cos(ang)
    sin = jnp.sin(ang)
    c_tab = jnp.repeat(cos, 2, axis=1)
    sg_tab = jnp.stack([-sin, sin], axis=-1).reshape(S, ROPE_DIM)
    pad = jnp.zeros((S, HEAD_DIM - ROPE_DIM), jnp.float32)
    c_tab = jnp.concatenate([c_tab, 1.0 + pad], axis=1)
    sg_tab = jnp.concatenate([sg_tab, pad], axis=1)
    return c_tab, sg_tab


def _kq_body(x_ref, wk_ref, ww_ref, c_ref, sg_ref, ln_w_ref, ln_b_ref,
             k_out, hw_out):
    x = x_ref[...]
    k = jnp.dot(x, wk_ref[...], preferred_element_type=jnp.float32)
    mu = jnp.mean(k, axis=-1, keepdims=True)
    var = jnp.var(k, axis=-1, keepdims=True)
    k = (k - mu) / jnp.sqrt(var + 1e-6) * ln_w_ref[...] + ln_b_ref[...]
    k = _rope_roll(k, c_ref[...], sg_ref[...])
    k = _fwht_roll(k) * (HEAD_DIM ** -0.5)
    k_out[...] = _quant(k)
    hw_out[...] = jnp.dot(x, ww_ref[...], preferred_element_type=jnp.float32)


def _q_body(qr_ref, wq_ref, c_ref, sg_ref, q_out):
    q = jnp.dot(qr_ref[...], wq_ref[...], preferred_element_type=jnp.float32)
    q = q.reshape(q.shape[0], N_HEADS, HEAD_DIM)
    q = _rope_roll(q, c_ref[...][:, None, :], sg_ref[...][:, None, :])
    q = _fwht_roll(q) * (HEAD_DIM ** -0.5)
    q = _quant(q)
    q_out[...] = q.reshape(q.shape[0], N_HEADS * HEAD_DIM)


_SBLK = 256


def _score_body(q_ref, k_ref, hw_ref, keys_out):
    i = pl.program_id(0)
    q = q_ref[...].reshape(_SBLK, N_HEADS, HEAD_DIM)
    k = k_ref[...]
    hw = hw_ref[...]
    acc = jnp.zeros((_SBLK, S), jnp.float32)
    for h in range(N_HEADS):
        sc = lax.dot_general(q[:, h, :], k, (((1,), (1,)), ((), ())),
                             preferred_element_type=jnp.float32,
                             precision=lax.Precision.HIGHEST)
        acc = acc + jnp.maximum(sc, 0.0) * hw[:, h:h + 1]
    acc = acc * (HEAD_DIM ** -0.5)
    srow = i * _SBLK + lax.broadcasted_iota(jnp.int32, (_SBLK, S), 0)
    tcol = lax.broadcasted_iota(jnp.int32, (_SBLK, S), 1)
    acc = jnp.where(srow >= tcol, acc, -1e30)
    keys_out[...] = _to_keys(acc)


def _scores_tc(x, qr, wq_b, wk, ln_w, ln_b, w_weights, position_ids):
    c_tab, sg_tab = _rope_tables(position_ids)
    ln_w2 = ln_w.reshape(1, HEAD_DIM)
    ln_b2 = ln_b.reshape(1, HEAD_DIM)
    kb = 512
    k_hat, hw = pl.pallas_call(
        _kq_body,
        grid=(S // kb,),
        in_specs=[
            pl.BlockSpec((kb, DIM), lambda i: (i, 0)),
            pl.BlockSpec((DIM, HEAD_DIM), lambda i: (0, 0)),
            pl.BlockSpec((DIM, N_HEADS), lambda i: (0, 0)),
            pl.BlockSpec((kb, HEAD_DIM), lambda i: (i, 0)),
            pl.BlockSpec((kb, HEAD_DIM), lambda i: (i, 0)),
            pl.BlockSpec((1, HEAD_DIM), lambda i: (0, 0)),
            pl.BlockSpec((1, HEAD_DIM), lambda i: (0, 0)),
        ],
        out_specs=[
            pl.BlockSpec((kb, HEAD_DIM), lambda i: (i, 0)),
            pl.BlockSpec((kb, N_HEADS), lambda i: (i, 0)),
        ],
        out_shape=[
            jax.ShapeDtypeStruct((S, HEAD_DIM), jnp.float32),
            jax.ShapeDtypeStruct((S, N_HEADS), jnp.float32),
        ],
    )(x, wk, w_weights, c_tab, sg_tab, ln_w2, ln_b2)

    qb = 512
    q_hat = pl.pallas_call(
        _q_body,
        grid=(S // qb,),
        in_specs=[
            pl.BlockSpec((qb, Q_LORA_RANK), lambda i: (i, 0)),
            pl.BlockSpec((Q_LORA_RANK, N_HEADS * HEAD_DIM), lambda i: (0, 0)),
            pl.BlockSpec((qb, HEAD_DIM), lambda i: (i, 0)),
            pl.BlockSpec((qb, HEAD_DIM), lambda i: (i, 0)),
        ],
        out_specs=pl.BlockSpec((qb, N_HEADS * HEAD_DIM), lambda i: (i, 0)),
        out_shape=jax.ShapeDtypeStruct((S, N_HEADS * HEAD_DIM), jnp.float32),
    )(qr, wq_b, c_tab, sg_tab)

    keys = pl.pallas_call(
        _score_body,
        grid=(S // _SBLK,),
        in_specs=[
            pl.BlockSpec((_SBLK, N_HEADS * HEAD_DIM), lambda i: (i, 0)),
            pl.BlockSpec((S, HEAD_DIM), lambda i: (0, 0)),
            pl.BlockSpec((_SBLK, N_HEADS), lambda i: (i, 0)),
        ],
        out_specs=pl.BlockSpec((_SBLK, S), lambda i: (i, 0)),
        out_shape=jax.ShapeDtypeStruct((S, S), jnp.int32),
    )(q_hat, k_hat, hw)
    return keys


def _sc_argsort(keys):
    mesh = plsc.VectorSubcoreMesh(core_axis_name="c", subcore_axis_name="s")
    B = 8

    @functools.partial(
        pl.kernel,
        out_type=jax.ShapeDtypeStruct((S, S), jnp.int32),
        mesh=mesh,
        compiler_params=pltpu.CompilerParams(needs_layout_passes=False),
        scratch_types=(
            [pltpu.VMEM((2080,), jnp.int32) for _ in range(B)]
            + [pltpu.VMEM((2080,), jnp.int32) for _ in range(4 * B)]
            + [pltpu.VMEM((64 * 16,), jnp.int32) for _ in range(B)]
            + [pltpu.VMEM((2080,), jnp.int32) for _ in range(B)]
        ),
    )
    def sort_kernel(keys_hbm, out_hbm, *scr):
        rowbufs = scr[0:B]
        ka = scr[B:2 * B]
        va = scr[2 * B:3 * B]
        kb = scr[3 * B:4 * B]
        vb = scr[4 * B:5 * B]
        hists = scr[5 * B:6 * B]
        outbufs = scr[6 * B:7 * B]
        a_bufs = tuple(zip(ka, va))
        b_bufs = tuple(zip(kb, vb))
        src0 = tuple((rb, None) for rb in rowbufs)
        wid = lax.axis_index("s") * _NC + lax.axis_index("c")
        lanes = lax.iota(jnp.int32, 16)
        ones = jnp.ones((16,), jnp.int32)
        zeros = jnp.zeros((16,), jnp.int32)

        def do_pass(p, shift, bins, nvo, srcs, dsts):
            def zstep(i, c):
                for j in range(B):
                    hists[j][pl.ds(i * 16, 16)] = zeros
                return c

            lax.fori_loop(0, bins, zstep, 0)

            def hstep(v, c):
                for j in range(B):
                    key = plsc.load_gather(srcs[j][0], [v + nvo * lanes])
                    d = lax.shift_right_logical(key, shift) & 63
                    plsc.addupdate_scatter(hists[j], [d * 16 + lanes], ones)
                return c

            lax.fori_loop(0, nvo, hstep, 0)

            def pstep(i, carries):
                outs = []
                for j in range(B):
                    h = hists[j][pl.ds(i * 16, 16)]
                    inc = plsc.cumsum(h)
                    hists[j][pl.ds(i * 16, 16)] = inc - h + carries[j]
                    outs.append(carries[j] + inc[15])
                return tuple(outs)

            lax.fori_loop(0, bins, pstep, (jnp.int32(0),) * B)

            def mstep(v, c):
                for j in range(B):
                    ii = v + nvo * lanes
                    key = plsc.load_gather(srcs[j][0], [ii])
                    if p == 0:
                        val = ii
                    else:
                        val = plsc.load_gather(srcs[j][1], [ii])
                    d = lax.shift_right_logical(key, shift) & 63
                    ih = d * 16 + lanes
                    r = plsc.load_gather(hists[j], [ih])
                    plsc.addupdate_scatter(hists[j], [ih], ones)
                    if p == 5:
                        plsc.store_scatter(outbufs[j], [r], val)
                    else:
                        plsc.store_scatter(dsts[j][0], [r], key)
                        plsc.store_scatter(dsts[j][1], [r], val)
                return c

            lax.fori_loop(0, nvo, mstep, 0)

        def row_body(ri, c):
            rows = tuple(wid + _NW * (B * ri + j) for j in range(B))
            nvo = lax.shift_right_logical(rows[B - 1] + 16, 4) | 1
            for j in range(B):
                pltpu.sync_copy(keys_hbm.at[rows[j]],
                                rowbufs[j].at[pl.ds(0, S)])

            def fill(v, c):
                for j in range(B):
                    outbufs[j][pl.ds(v * 16, 16)] = v * 16 + lanes
                return c

            lax.fori_loop(nvo, _NV, fill, 0)

            def padstep(v, c):
                for j in range(B):
                    a = v * 16 + lanes
                    rb = rowbufs[j]
                    orig = rb[pl.ds(v * 16, 16)]
                    rb[pl.ds(v * 16, 16)] = jnp.where(
                        a < rows[j] + 1, orig, jnp.int32(-65536) + a)
                return c

            lax.fori_loop(lax.shift_right_logical(rows[0] + 1, 4), nvo,
                          padstep, 0)

            do_pass(0, 0, 64, nvo, src0, a_bufs)
            do_pass(1, 6, 64, nvo, a_bufs, b_bufs)
            do_pass(2, 12, 64, nvo, b_bufs, a_bufs)
            do_pass(3, 18, 64, nvo, a_bufs, b_bufs)
            do_pass(4, 24, 64, nvo, b_bufs, a_bufs)
            do_pass(5, 30, 4, nvo, a_bufs, None)
            for j in range(B):
                pltpu.sync_copy(outbufs[j].at[pl.ds(0, S)],
                                out_hbm.at[rows[j]])
            return c

        lax.fori_loop(0, _ROWS_PER_W // B, row_body, 0)

    return sort_kernel(keys)


def kernel(x, qr, wq_b, wk, ln_w, ln_b, w_weights, position_ids):
    scores = _scores(x, qr, wq_b, wk, ln_w, ln_b, w_weights, position_ids)
    keys = _to_keys(scores)
    return _sc_argsort(keys)


# speedup vs baseline: 1.9716x; 1.0016x over previous
import functools

import jax
import jax.numpy as jnp
from jax import lax
from jax.experimental import pallas as pl
from jax.experimental.pallas import tpu as pltpu
from jax.experimental.pallas import tpu_sc as plsc

S = 2048
DIM = 2048
Q_LORA_RANK = 1536
N_HEADS = 16
HEAD_DIM = 128
ROPE_DIM = 64
INDEX_TOPK = 2048
BLOCK = 128

_NC = 2
_NS = 16
_NW = _NC * _NS
_NV = S // 16
_ROWS_PER_W = S // _NW


def _fwht(x):
    d = x.shape[-1]
    h = 1
    while h < d:
        x = x.reshape(x.shape[:-1] + (d // (2 * h), 2, h))
        a = x[..., 0, :]
        b = x[..., 1, :]
        x = jnp.stack([a + b, a - b], axis=-2)
        x = x.reshape(x.shape[:-3] + (d,))
        h *= 2
    return x


def _rope_interleaved(x, cos, sin, rot_end):
    rot = x[..., :rot_end]
    rest = x[..., rot_end:]
    x1 = rot[..., 0::2]
    x2 = rot[..., 1::2]
    o1 = x1 * cos - x2 * sin
    o2 = x1 * sin + x2 * cos
    out = jnp.stack([o1, o2], axis=-1).reshape(rot.shape)
    return jnp.concatenate([out, rest], axis=-1)


def _block_quant_dequant(x, block=BLOCK):
    shp = x.shape
    xb = x.reshape(shp[:-1] + (shp[-1] // block, block))
    amax = jnp.max(jnp.abs(xb), axis=-1, keepdims=True)
    scale = jnp.maximum(amax, 1e-4) / 448.0
    q = jnp.clip(xb / scale, -448.0, 448.0)
    return (q * scale).reshape(shp)


def _scores(x, qr, wq_b, wk, ln_w, ln_b, w_weights, position_ids):
    softmax_scale = HEAD_DIM ** -0.5
    q = (qr @ wq_b).reshape(S, N_HEADS, HEAD_DIM)
    k = x @ wk
    mu = jnp.mean(k, axis=-1, keepdims=True)
    var = jnp.var(k, axis=-1, keepdims=True)
    k = (k - mu) / jnp.sqrt(var + 1e-6) * ln_w + ln_b
    inv_freq = 1.0 / (10000.0 ** (jnp.arange(0, ROPE_DIM, 2, dtype=jnp.float32) / ROPE_DIM))
    ang = position_ids.astype(jnp.float32)[:, None] * inv_freq[None, :]
    cos = jnp.cos(ang)
    sin = jnp.sin(ang)
    q = _rope_interleaved(q, cos[:, None, :], sin[:, None, :], ROPE_DIM)
    k = _rope_interleaved(k, cos, sin, ROPE_DIM)
    q = _fwht(q) * (HEAD_DIM ** -0.5)
    k = _fwht(k) * (HEAD_DIM ** -0.5)
    q = _block_quant_dequant(q)
    k = _block_quant_dequant(k)
    head_w = x @ w_weights
    scores = jnp.einsum('shd,td->sht', q, k)
    scores = jax.nn.relu(scores)
    scores = jnp.einsum('sht,sh->st', scores, head_w) * softmax_scale
    causal = position_ids[:, None] >= position_ids[None, :]
    return jnp.where(causal, scores, -1e30)


def _to_keys(scores):
    s = scores + 0.0
    u = lax.bitcast_convert_type(s, jnp.int32)
    return jnp.where(s < 0.0, u, ~u & jnp.int32(0x7FFFFFFF))


def _lane_iota(shape):
    return lax.broadcasted_iota(jnp.int32, shape, len(shape) - 1)


def _rope_roll(x, c_tab, sg_tab):
    li = _lane_iota(x.shape)
    even = (li & 1) == 0
    ax = len(x.shape) - 1
    xs = jnp.where(even, pltpu.roll(x, HEAD_DIM - 1, ax),
                   pltpu.roll(x, 1, ax))
    ro = x * c_tab + xs * sg_tab
    return jnp.where(li < ROPE_DIM, ro, x)


def _fwht_roll(x):
    li = _lane_iota(x.shape)
    ax = len(x.shape) - 1
    for h in (1, 2, 4, 8, 16, 32, 64):
        even = (li & h) == 0
        partner = jnp.where(even, pltpu.roll(x, HEAD_DIM - h, ax),
                            pltpu.roll(x, h, ax))
        x = jnp.where(even, x + partner, partner - x)
    return x


def _quant(x):
    amax = jnp.max(jnp.abs(x), axis=-1, keepdims=True)
    scale = jnp.maximum(amax, 1e-4) / 448.0
    return jnp.clip(x / scale, -448.0, 448.0) * scale


def _rope_tables(position_ids):
    inv_freq = 1.0 / (10000.0 ** (jnp.arange(0, ROPE_DIM, 2, dtype=jnp.float32) / ROPE_DIM))
    ang = position_ids.astype(jnp.float32)[:, None] * inv_freq[None, :]
    cos = jnp.cos(ang)
    sin = jnp.sin(ang)
    c_tab = jnp.repeat(cos, 2, axis=1)
    sg_tab = jnp.stack([-sin, sin], axis=-1).reshape(S, ROPE_DIM)
    pad = jnp.zeros((S, HEAD_DIM - ROPE_DIM), jnp.float32)
    c_tab = jnp.concatenate([c_tab, 1.0 + pad], axis=1)
    sg_tab = jnp.concatenate([sg_tab, pad], axis=1)
    return c_tab, sg_tab


def _kq_body(x_ref, wk_ref, ww_ref, c_ref, sg_ref, ln_w_ref, ln_b_ref,
             k_out, hw_out):
    x = x_ref[...]
    k = jnp.dot(x, wk_ref[...], preferred_element_type=jnp.float32)
    mu = jnp.mean(k, axis=-1, keepdims=True)
    var = jnp.var(k, axis=-1, keepdims=True)
    k = (k - mu) / jnp.sqrt(var + 1e-6) * ln_w_ref[...] + ln_b_ref[...]
    k = _rope_roll(k, c_ref[...], sg_ref[...])
    k = _fwht_roll(k) * (HEAD_DIM ** -0.5)
    k_out[...] = _quant(k)
    hw_out[...] = jnp.dot(x, ww_ref[...], preferred_element_type=jnp.float32)


def _q_body(qr_ref, wq_ref, c_ref, sg_ref, q_out):
    q = jnp.dot(qr_ref[...], wq_ref[...], preferred_element_type=jnp.float32)
    q = q.reshape(q.shape[0], N_HEADS, HEAD_DIM)
    q = _rope_roll(q, c_ref[...][:, None, :], sg_ref[...][:, None, :])
    q = _fwht_roll(q) * (HEAD_DIM ** -0.5)
    q = _quant(q)
    q_out[...] = q.reshape(q.shape[0], N_HEADS * HEAD_DIM)


_SBLK = 256


def _score_body(q_ref, k_ref, hw_ref, keys_out):
    i = pl.program_id(0)
    q = q_ref[...].reshape(_SBLK, N_HEADS, HEAD_DIM)
    k = k_ref[...]
    hw = hw_ref[...]
    acc = jnp.zeros((_SBLK, S), jnp.float32)
    for h in range(N_HEADS):
        sc = lax.dot_general(q[:, h, :], k, (((1,), (1,)), ((), ())),
                             preferred_element_type=jnp.float32,
                             precision=lax.Precision.HIGHEST)
        acc = acc + jnp.maximum(sc, 0.0) * hw[:, h:h + 1]
    acc = acc * (HEAD_DIM ** -0.5)
    srow = i * _SBLK + lax.broadcasted_iota(jnp.int32, (_SBLK, S), 0)
    tcol = lax.broadcasted_iota(jnp.int32, (_SBLK, S), 1)
    acc = jnp.where(srow >= tcol, acc, -1e30)
    keys_out[...] = _to_keys(acc)


def _scores_tc(x, qr, wq_b, wk, ln_w, ln_b, w_weights, position_ids):
    c_tab, sg_tab = _rope_tables(position_ids)
    ln_w2 = ln_w.reshape(1, HEAD_DIM)
    ln_b2 = ln_b.reshape(1, HEAD_DIM)
    kb = 512
    k_hat, hw = pl.pallas_call(
        _kq_body,
        grid=(S // kb,),
        in_specs=[
            pl.BlockSpec((kb, DIM), lambda i: (i, 0)),
            pl.BlockSpec((DIM, HEAD_DIM), lambda i: (0, 0)),
            pl.BlockSpec((DIM, N_HEADS), lambda i: (0, 0)),
            pl.BlockSpec((kb, HEAD_DIM), lambda i: (i, 0)),
            pl.BlockSpec((kb, HEAD_DIM), lambda i: (i, 0)),
            pl.BlockSpec((1, HEAD_DIM), lambda i: (0, 0)),
            pl.BlockSpec((1, HEAD_DIM), lambda i: (0, 0)),
        ],
        out_specs=[
            pl.BlockSpec((kb, HEAD_DIM), lambda i: (i, 0)),
            pl.BlockSpec((kb, N_HEADS), lambda i: (i, 0)),
        ],
        out_shape=[
            jax.ShapeDtypeStruct((S, HEAD_DIM), jnp.float32),
            jax.ShapeDtypeStruct((S, N_HEADS), jnp.float32),
        ],
    )(x, wk, w_weights, c_tab, sg_tab, ln_w2, ln_b2)

    qb = 512
    q_hat = pl.pallas_call(
        _q_body,
        grid=(S // qb,),
        in_specs=[
            pl.BlockSpec((qb, Q_LORA_RANK), lambda i: (i, 0)),
            pl.BlockSpec((Q_LORA_RANK, N_HEADS * HEAD_DIM), lambda i: (0, 0)),
            pl.BlockSpec((qb, HEAD_DIM), lambda i: (i, 0)),
            pl.BlockSpec((qb, HEAD_DIM), lambda i: (i, 0)),
        ],
        out_specs=pl.BlockSpec((qb, N_HEADS * HEAD_DIM), lambda i: (i, 0)),
        out_shape=jax.ShapeDtypeStruct((S, N_HEADS * HEAD_DIM), jnp.float32),
    )(qr, wq_b, c_tab, sg_tab)

    keys = pl.pallas_call(
        _score_body,
        grid=(S // _SBLK,),
        in_specs=[
            pl.BlockSpec((_SBLK, N_HEADS * HEAD_DIM), lambda i: (i, 0)),
            pl.BlockSpec((S, HEAD_DIM), lambda i: (0, 0)),
            pl.BlockSpec((_SBLK, N_HEADS), lambda i: (i, 0)),
        ],
        out_specs=pl.BlockSpec((_SBLK, S), lambda i: (i, 0)),
        out_shape=jax.ShapeDtypeStruct((S, S), jnp.int32),
    )(q_hat, k_hat, hw)
    return keys


def _sc_argsort(keys):
    mesh = plsc.VectorSubcoreMesh(core_axis_name="c", subcore_axis_name="s")
    B = 8

    @functools.partial(
        pl.kernel,
        out_type=jax.ShapeDtypeStruct((S, S), jnp.int32),
        mesh=mesh,
        compiler_params=pltpu.CompilerParams(needs_layout_passes=False),
        scratch_types=(
            [pltpu.VMEM((2080,), jnp.int32) for _ in range(B)]
            + [pltpu.VMEM((2080,), jnp.int32) for _ in range(4 * B)]
            + [pltpu.VMEM((64 * 16,), jnp.int32) for _ in range(B)]
            + [pltpu.VMEM((2080,), jnp.int32) for _ in range(B)]
        ),
    )
    def sort_kernel(keys_hbm, out_hbm, *scr):
        rowbufs = scr[0:B]
        ka = scr[B:2 * B]
        va = scr[2 * B:3 * B]
        kb = scr[3 * B:4 * B]
        vb = scr[4 * B:5 * B]
        hists = scr[5 * B:6 * B]
        outbufs = scr[6 * B:7 * B]
        a_bufs = tuple(zip(ka, va))
        b_bufs = tuple(zip(kb, vb))
        src0 = tuple((rb, None) for rb in rowbufs)
        wid = lax.axis_index("s") * _NC + lax.axis_index("c")
        lanes = lax.iota(jnp.int32, 16)
        ones = jnp.ones((16,), jnp.int32)
        zeros = jnp.zeros((16,), jnp.int32)

        def do_pass(p, shift, bins, nvo, srcs, dsts):
            def zstep(i, c):
                for j in range(B):
                    hists[j][pl.ds(i * 16, 16)] = zeros
                return c

            lax.fori_loop(0, bins, zstep, 0)

            def hstep(v, c):
                ii = v + nvo * lanes
                for j in range(B):
                    key = plsc.load_gather(srcs[j][0], [ii])
                    d = lax.shift_right_logical(key, shift) & 63
                    plsc.addupdate_scatter(hists[j], [d * 16 + lanes], ones)
                return c

            lax.fori_loop(0, nvo, hstep, 0)

            def pstep(i, carries):
                outs = []
                for j in range(B):
                    h = hists[j][pl.ds(i * 16, 16)]
                    inc = plsc.cumsum(h)
                    hists[j][pl.ds(i * 16, 16)] = inc - h + carries[j]
                    outs.append(carries[j] + inc[15])
                return tuple(outs)

            lax.fori_loop(0, bins, pstep, (jnp.int32(0),) * B)

            def mstep(v, c):
                ii = v + nvo * lanes
                for j in range(B):
                    key = plsc.load_gather(srcs[j][0], [ii])
                    if p == 0:
                        val = ii
                    else:
                        val = plsc.load_gather(srcs[j][1], [ii])
                    d = lax.shift_right_logical(key, shift) & 63
                    ih = d * 16 + lanes
                    r = plsc.load_gather(hists[j], [ih])
                    plsc.addupdate_scatter(hists[j], [ih], ones)
                    if p == 5:
                        plsc.store_scatter(outbufs[j], [r], val)
                    else:
                        plsc.store_scatter(dsts[j][0], [r], key)
                        plsc.store_scatter(dsts[j][1], [r], val)
                return c

            lax.fori_loop(0, nvo, mstep, 0)

        def fill(v, c):
            for j in range(B):
                outbufs[j][pl.ds(v * 16, 16)] = v * 16 + lanes
            return c

        lax.fori_loop(0, _NV, fill, 0)

        def row_body(ri, c):
            rows = tuple(wid + _NW * (B * ri + j) for j in range(B))
            nvo = lax.shift_right_logical(rows[B - 1] + 16, 4) | 1
            for j in range(B):
                pltpu.sync_copy(keys_hbm.at[rows[j]],
                                rowbufs[j].at[pl.ds(0, S)])

            def padstep(v, c):
                for j in range(B):
                    a = v * 16 + lanes
                    rb = rowbufs[j]
                    orig = rb[pl.ds(v * 16, 16)]
                    rb[pl.ds(v * 16, 16)] = jnp.where(
                        a < rows[j] + 1, orig, jnp.int32(-65536) + a)
                return c

            lax.fori_loop(lax.shift_right_logical(rows[0] + 1, 4), nvo,
                          padstep, 0)

            do_pass(0, 0, 64, nvo, src0, a_bufs)
            do_pass(1, 6, 64, nvo, a_bufs, b_bufs)
            do_pass(2, 12, 64, nvo, b_bufs, a_bufs)
            do_pass(3, 18, 64, nvo, a_bufs, b_bufs)
            do_pass(4, 24, 64, nvo, b_bufs, a_bufs)
            do_pass(5, 30, 4, nvo, a_bufs, None)
            for j in range(B):
                pltpu.sync_copy(outbufs[j].at[pl.ds(0, S)],
                                out_hbm.at[rows[j]])
            return c

        lax.fori_loop(0, _ROWS_PER_W // B, row_body, 0)

    return sort_kernel(keys)


def kernel(x, qr, wq_b, wk, ln_w, ln_b, w_weights, position_ids):
    scores = _scores(x, qr, wq_b, wk, ln_w, ln_b, w_weights, position_ids)
    keys = _to_keys(scores)
    return _sc_argsort(keys)


# speedup vs baseline: 1.9739x; 1.0012x over previous
import functools

import jax
import jax.numpy as jnp
from jax import lax
from jax.experimental import pallas as pl
from jax.experimental.pallas import tpu as pltpu
from jax.experimental.pallas import tpu_sc as plsc

S = 2048
DIM = 2048
Q_LORA_RANK = 1536
N_HEADS = 16
HEAD_DIM = 128
ROPE_DIM = 64
INDEX_TOPK = 2048
BLOCK = 128

_NC = 2
_NS = 16
_NW = _NC * _NS
_NV = S // 16
_ROWS_PER_W = S // _NW


def _fwht(x):
    d = x.shape[-1]
    h = 1
    while h < d:
        x = x.reshape(x.shape[:-1] + (d // (2 * h), 2, h))
        a = x[..., 0, :]
        b = x[..., 1, :]
        x = jnp.stack([a + b, a - b], axis=-2)
        x = x.reshape(x.shape[:-3] + (d,))
        h *= 2
    return x


def _rope_interleaved(x, cos, sin, rot_end):
    rot = x[..., :rot_end]
    rest = x[..., rot_end:]
    x1 = rot[..., 0::2]
    x2 = rot[..., 1::2]
    o1 = x1 * cos - x2 * sin
    o2 = x1 * sin + x2 * cos
    out = jnp.stack([o1, o2], axis=-1).reshape(rot.shape)
    return jnp.concatenate([out, rest], axis=-1)


def _block_quant_dequant(x, block=BLOCK):
    shp = x.shape
    xb = x.reshape(shp[:-1] + (shp[-1] // block, block))
    amax = jnp.max(jnp.abs(xb), axis=-1, keepdims=True)
    scale = jnp.maximum(amax, 1e-4) / 448.0
    q = jnp.clip(xb / scale, -448.0, 448.0)
    return (q * scale).reshape(shp)


def _scores(x, qr, wq_b, wk, ln_w, ln_b, w_weights, position_ids):
    softmax_scale = HEAD_DIM ** -0.5
    q = (qr @ wq_b).reshape(S, N_HEADS, HEAD_DIM)
    k = x @ wk
    mu = jnp.mean(k, axis=-1, keepdims=True)
    var = jnp.var(k, axis=-1, keepdims=True)
    k = (k - mu) / jnp.sqrt(var + 1e-6) * ln_w + ln_b
    inv_freq = 1.0 / (10000.0 ** (jnp.arange(0, ROPE_DIM, 2, dtype=jnp.float32) / ROPE_DIM))
    ang = position_ids.astype(jnp.float32)[:, None] * inv_freq[None, :]
    cos = jnp.cos(ang)
    sin = jnp.sin(ang)
    q = _rope_interleaved(q, cos[:, None, :], sin[:, None, :], ROPE_DIM)
    k = _rope_interleaved(k, cos, sin, ROPE_DIM)
    q = _fwht(q) * (HEAD_DIM ** -0.5)
    k = _fwht(k) * (HEAD_DIM ** -0.5)
    q = _block_quant_dequant(q)
    k = _block_quant_dequant(k)
    head_w = x @ w_weights
    scores = jnp.einsum('shd,td->sht', q, k)
    scores = jax.nn.relu(scores)
    scores = jnp.einsum('sht,sh->st', scores, head_w) * softmax_scale
    causal = position_ids[:, None] >= position_ids[None, :]
    return jnp.where(causal, scores, -1e30)


def _to_keys(scores):
    s = scores + 0.0
    u = lax.bitcast_convert_type(s, jnp.int32)
    return jnp.where(s < 0.0, u, ~u & jnp.int32(0x7FFFFFFF))


def _sc_argsort(keys):
    mesh = plsc.VectorSubcoreMesh(core_axis_name="c", subcore_axis_name="s")
    B = 8

    @functools.partial(
        pl.kernel,
        out_type=jax.ShapeDtypeStruct((S, S), jnp.int32),
        mesh=mesh,
        compiler_params=pltpu.CompilerParams(needs_layout_passes=False),
        scratch_types=(
            [pltpu.VMEM((2080,), jnp.int32) for _ in range(B)]
            + [pltpu.VMEM((2080,), jnp.int32) for _ in range(4 * B)]
            + [pltpu.VMEM((64 * 16,), jnp.int32) for _ in range(B)]
            + [pltpu.VMEM((2080,), jnp.int32) for _ in range(B)]
        ),
    )
    def sort_kernel(keys_hbm, out_hbm, *scr):
        rowbufs = scr[0:B]
        ka = scr[B:2 * B]
        va = scr[2 * B:3 * B]
        kb = scr[3 * B:4 * B]
        vb = scr[4 * B:5 * B]
        hists = scr[5 * B:6 * B]
        outbufs = scr[6 * B:7 * B]
        a_bufs = tuple(zip(ka, va))
        b_bufs = tuple(zip(kb, vb))
        src0 = tuple((rb, None) for rb in rowbufs)
        wid = lax.axis_index("s") * _NC + lax.axis_index("c")
        lanes = lax.iota(jnp.int32, 16)
        ones = jnp.ones((16,), jnp.int32)
        zeros = jnp.zeros((16,), jnp.int32)

        def do_pass(p, shift, bins, nvo, srcs, dsts):
            def zstep(i, c):
                for j in range(B):
                    hists[j][pl.ds(i * 16, 16)] = zeros
                return c

            lax.fori_loop(0, bins, zstep, 0)

            def hstep(v, c):
                ii = v + nvo * lanes
                for j in range(B):
                    key = plsc.load_gather(srcs[j][0], [ii])
                    d = lax.shift_right_logical(key, shift) & 63
                    plsc.addupdate_scatter(hists[j], [d * 16 + lanes], ones)
                return c

            lax.fori_loop(0, nvo, hstep, 0)

            def pstep(i, carries):
                outs = []
                for j in range(B):
                    h = hists[j][pl.ds(i * 16, 16)]
                    inc = plsc.cumsum(h)
                    hists[j][pl.ds(i * 16, 16)] = inc - h + carries[j]
                    outs.append(carries[j] + inc[15])
                return tuple(outs)

            lax.fori_loop(0, bins, pstep, (jnp.int32(0),) * B)

            def mstep(v, c):
                ii = v + nvo * lanes
                for j in range(B):
                    key = plsc.load_gather(srcs[j][0], [ii])
                    if p == 0:
                        val = ii
                    else:
                        val = plsc.load_gather(srcs[j][1], [ii])
                    d = lax.shift_right_logical(key, shift) & 63
                    ih = d * 16 + lanes
                    r = plsc.load_gather(hists[j], [ih])
                    plsc.addupdate_scatter(hists[j], [ih], ones)
                    if p == 5:
                        plsc.store_scatter(outbufs[j], [r], val)
                    else:
                        plsc.store_scatter(dsts[j][0], [r], key)
                        plsc.store_scatter(dsts[j][1], [r], val)
                return c

            lax.fori_loop(0, nvo, mstep, 0)

        def fill(v, c):
            for j in range(B):
                outbufs[j][pl.ds(v * 16, 16)] = v * 16 + lanes
            return c

        lax.fori_loop(0, _NV, fill, 0)

        def row_body(ri, c):
            rows = tuple(wid + _NW * (B * ri + j) for j in range(B))
            nvo = lax.shift_right_logical(rows[B - 1] + 16, 4) | 1
            for j in range(B):
                pltpu.sync_copy(keys_hbm.at[rows[j]],
                                rowbufs[j].at[pl.ds(0, S)])

            def padstep(v, c):
                for j in range(B):
                    a = v * 16 + lanes
                    rb = rowbufs[j]
                    orig = rb[pl.ds(v * 16, 16)]
                    rb[pl.ds(v * 16, 16)] = jnp.where(
                        a < rows[j] + 1, orig, jnp.int32(-65536) + a)
                return c

            lax.fori_loop(lax.shift_right_logical(rows[0] + 1, 4), nvo,
                          padstep, 0)

            do_pass(0, 0, 64, nvo, src0, a_bufs)
            do_pass(1, 6, 64, nvo, a_bufs, b_bufs)
            do_pass(2, 12, 64, nvo, b_bufs, a_bufs)
            do_pass(3, 18, 64, nvo, a_bufs, b_bufs)
            do_pass(4, 24, 64, nvo, b_bufs, a_bufs)
            do_pass(5, 30, 4, nvo, a_bufs, None)
            for j in range(B):
                pltpu.sync_copy(outbufs[j].at[pl.ds(0, S)],
                                out_hbm.at[rows[j]])
            return c

        lax.fori_loop(0, _ROWS_PER_W // B, row_body, 0)

    return sort_kernel(keys)


def kernel(x, qr, wq_b, wk, ln_w, ln_b, w_weights, position_ids):
    scores = _scores(x, qr, wq_b, wk, ln_w, ln_b, w_weights, position_ids)
    keys = _to_keys(scores)
    return _sc_argsort(keys)
